# baseline (speedup 1.0000x reference)
import functools

import jax
import jax.numpy as jnp
import numpy as np
from jax import lax
from jax.experimental import pallas as pl
from jax.experimental.pallas import tpu as pltpu
from jax.experimental.pallas import tpu_sc as plsc

B, N = 4, 6
ogfH, ogfW = 256, 704
DS = 16
fH, fW = ogfH // DS, ogfW // DS
HW = fH * fW
D, C = 48, 64
NCAM = B * N
NX, NY = 200, 200
NVOX = NX * NY
VPAD = 40960
OFFX = np.float32(-50.0)
OFFY = np.float32(-50.0)
OFFZ = np.float32(-10.0)
DXX = np.float32(0.5)
DXY = np.float32(0.5)
DXZ = np.float32(20.0)
BXV = np.array([-49.75, -49.75, 0.0], dtype=np.float32)
DXV = np.array([0.5, 0.5, 20.0], dtype=np.float32)


def _create_frustum():
    ds = jnp.broadcast_to(jnp.linspace(2.0, 50.0, D, dtype=jnp.float32).reshape(D, 1, 1), (D, fH, fW))
    xs = jnp.broadcast_to(jnp.linspace(0.0, ogfW - 1.0, fW, dtype=jnp.float32).reshape(1, 1, fW), (D, fH, fW))
    ys = jnp.broadcast_to(jnp.linspace(0.0, ogfH - 1.0, fH, dtype=jnp.float32).reshape(1, fH, 1), (D, fH, fW))
    return jnp.stack([xs, ys, ds], axis=-1)


def _encode_body(img_ref, w_ref, b_ref, pool_ref, depth_ref, feat_ref):
    img = img_ref[0]
    a = jnp.sum(img.reshape(3, fH, DS, ogfW), axis=2)
    a2 = a.reshape(3 * fH, ogfW)
    xi44 = jnp.dot(a2, pool_ref[...], preferred_element_type=jnp.float32)
    xi3 = xi44.reshape(3, fH, fW)

    w = w_ref[...]
    bb = b_ref[...]
    out_d = (bb[0][:, None, None]
             + w[0][:, None, None] * xi3[0][None, :, :]
             + w[1][:, None, None] * xi3[1][None, :, :]
             + w[2][:, None, None] * xi3[2][None, :, :])
    logits = out_d[:D]
    m = jnp.max(logits, axis=0, keepdims=True)
    e = jnp.exp(logits - m)
    depth = e / jnp.sum(e, axis=0, keepdims=True)
    depth_ref[0] = depth

    out_c = (bb[0][None, None, :]
             + xi3[0][:, :, None] * w[0][None, None, :]
             + xi3[1][:, :, None] * w[1][None, None, :]
             + xi3[2][:, :, None] * w[2][None, None, :])
    feat_ref[0] = out_c[:, :, D:D + C]


def _encode(imgs, W_enc, b_enc, poolW):
    grid = (NCAM,)
    return pl.pallas_call(
        _encode_body,
        grid=grid,
        in_specs=[
            pl.BlockSpec((1, 3, ogfH, ogfW), lambda k: (k, 0, 0, 0)),
            pl.BlockSpec((3, D + C), lambda k: (0, 0)),
            pl.BlockSpec((1, D + C), lambda k: (0, 0)),
            pl.BlockSpec((ogfW, fW), lambda k: (0, 0)),
        ],
        out_specs=[
            pl.BlockSpec((1, D, fH, fW), lambda k: (k, 0, 0, 0)),
            pl.BlockSpec((1, fH, fW, C), lambda k: (k, 0, 0, 0)),
        ],
        out_shape=[
            jax.ShapeDtypeStruct((NCAM, D, fH, fW), jnp.float32),
            jax.ShapeDtypeStruct((NCAM, fH, fW, C), jnp.float32),
        ],
    )(imgs, W_enc, b_enc, poolW)


NSLAB = N * D
SPT = NSLAB // 16
RPT = VPAD // 16
FWORDS = HW * 16


def _bcast16(vec, l):
    dn = lax.GatherDimensionNumbers(offset_dims=(), collapsed_slice_dims=(0,),
                                    start_index_map=(0,))
    idxv = jnp.full((16,), l, jnp.int32)
    return lax.gather(vec, idxv[:, None], dn, (1,),
                      mode=lax.GatherScatterMode.PROMISE_IN_BOUNDS)


def _splat_body(depth_hbm, feat_hbm, idxp_hbm, zeros_hbm, out_hbm,
                featv, depv, ibuf, vbuf, zbuf, cbuf, tbuf, dsem, ssem, acc):
    cid = lax.axis_index("c")
    tid = lax.axis_index("s")
    base = tid * RPT
    iota16 = lax.iota(jnp.int32, 16)
    pltpu.sync_copy(zeros_hbm, zbuf)
    pltpu.sync_copy(zeros_hbm.at[pl.ds(0, 64)], vbuf.at[pl.ds(704, 64)])

    def do_slab(j, b):
        s = b * NSLAB + tid * SPT + j
        h1 = pltpu.async_copy(depth_hbm.at[s], depv, dsem)
        h2 = pltpu.async_copy(idxp_hbm.at[s], ibuf, dsem)
        h1.wait()
        h2.wait()

        def grp(g, carry):
            for q in range(4):
                dvec = depv[pl.ds(g * 64 + q * 16, 16)]
                for l in range(16):
                    dj = _bcast16(dvec, l)
                    r = g * 64 + q * 16 + l
                    vbuf[r] = featv[pl.ds(r * 16, 16)] * dj
            return carry

        lax.fori_loop(0, HW // 64, grp, 0)
        hs = [pltpu.async_copy(vbuf.at[pl.ds(j6 * 128, 128)],
                               acc.at[ibuf.at[j6]], ssem, add=True)
              for j6 in range(6)]
        for hh in hs:
            hh.wait()
        return b

    def do_round(r, carry):
        b = cid * 2 + r // 4
        h = r % 4
        if True:
            pltpu.sync_copy(zbuf, acc.at[pl.ds(base, 1280)])
            pltpu.sync_copy(zbuf, acc.at[pl.ds(base + 1280, 1280)])
            plsc.subcore_barrier()
            s_lo = tid * SPT
            nA = s_lo // 48
            mA = jnp.minimum((nA + 1) * 48 - s_lo, SPT)
            pltpu.sync_copy(feat_hbm.at[h, b * N + nA], featv)
            lax.fori_loop(0, mA, do_slab, b)

            @pl.when(mA < SPT)
            def _():
                pltpu.sync_copy(feat_hbm.at[h, b * N + nA + 1], featv)

            lax.fori_loop(mA, SPT, do_slab, b)
            plsc.subcore_barrier()

            def xp_chunk(v0, nrows):
                pltpu.sync_copy(acc.at[pl.ds(v0, nrows)], cbuf.at[pl.ds(0, nrows)])

                def xg(g, carry):
                    rowv = g * 16 + iota16
                    for cc in range(16):
                        vv = plsc.load_gather(cbuf, [rowv, jnp.full((16,), cc, jnp.int32)])
                        tbuf[cc, pl.ds(g * 16, 16)] = vv
                    return carry

                lax.fori_loop(0, nrows // 16, xg, 0)
                pltpu.sync_copy(tbuf.at[:, pl.ds(0, nrows)],
                                out_hbm.at[b, pl.ds(h * 16, 16), pl.ds(v0, nrows)])

            for c4 in range(4):
                @pl.when(base + (c4 + 1) * 640 <= NVOX)
                def _():
                    xp_chunk(base + c4 * 640, 640)

            @pl.when(tid == 15)
            def _():
                xp_chunk(39680, 320)

            plsc.subcore_barrier()
        return carry

    lax.fori_loop(0, 8, do_round, 0)


def _splat(depth_r, feat_r, idxp, zeros):
    mesh = plsc.VectorSubcoreMesh(core_axis_name="c", subcore_axis_name="s")
    f = pl.kernel(
        _splat_body,
        out_type=jax.ShapeDtypeStruct((B, C, NVOX), jnp.float32),
        mesh=mesh,
        scratch_types=[
            pltpu.VMEM((FWORDS,), jnp.float32),
            pltpu.VMEM((HW,), jnp.float32),
            pltpu.VMEM((6, 128), jnp.int32),
            pltpu.VMEM((768, 16), jnp.float32),
            pltpu.VMEM((1280, 16), jnp.float32),
            pltpu.VMEM((640, 16), jnp.float32),
            pltpu.VMEM((16, 640), jnp.float32),
            pltpu.SemaphoreType.DMA,
            pltpu.SemaphoreType.DMA,
            pltpu.VMEM_SHARED((VPAD, 16), jnp.float32),
        ],
        compiler_params=pltpu.CompilerParams(use_tc_tiling_on_sc=False, needs_layout_passes=False),
    )
    return f(depth_r, feat_r, idxp, zeros)


def kernel(imgs, rots, trans, intrins, post_rots, post_trans, W_enc, b_enc):
    frustum = _create_frustum()
    points = frustum[None, None] - post_trans.reshape(B, N, 1, 1, 1, 3)
    inv_post = jnp.linalg.inv(post_rots)
    points = jnp.einsum('bnij,bndhwj->bndhwi', inv_post, points)
    points = jnp.concatenate([points[..., :2] * points[..., 2:3], points[..., 2:3]], axis=-1)
    combine = jnp.einsum('bnij,bnjk->bnik', rots, jnp.linalg.inv(intrins))
    geom = jnp.einsum('bnij,bndhwj->bndhwi', combine, points) + trans.reshape(B, N, 1, 1, 1, 3)
    gf = ((geom - (jnp.asarray(BXV) - jnp.asarray(DXV) / 2.0)) / jnp.asarray(DXV)).astype(jnp.int32)
    ix, iy, iz = gf[..., 0], gf[..., 1], gf[..., 2]
    kept = (ix >= 0) & (ix < NX) & (iy >= 0) & (iy < NY) & (iz >= 0) & (iz < 1)
    lane = lax.broadcasted_iota(jnp.int32, (B, N, D, fH, fW), 4)
    idx = jnp.where(kept, iy * NX + ix, NVOX + (lane & 7))

    poolW = np.zeros((ogfW, fW), np.float32)
    for wq in range(fW):
        poolW[wq * DS:(wq + 1) * DS, wq] = 1.0 / (DS * DS)
    poolW = jnp.asarray(poolW)

    depth, feat = _encode(imgs.reshape(NCAM, 3, ogfH, ogfW), W_enc,
                          b_enc.reshape(1, D + C), poolW)

    depth_r = depth.reshape(NCAM * D, HW)
    feat_r = feat.reshape(NCAM, HW, 4, 16).transpose(2, 0, 1, 3).reshape(4, NCAM, FWORDS)
    idx_r = idx.reshape(NCAM * D, HW)
    tail = NVOX + (lax.broadcasted_iota(jnp.int32, (NCAM * D, 64), 1) & 7)
    idxp = jnp.concatenate([idx_r, tail], axis=1).reshape(NCAM * D, 6, 128)
    zeros = jnp.zeros((1280, 16), jnp.float32)
    out = _splat(depth_r, feat_r, idxp, zeros)
    return out.reshape(B, C, NY, NX)


# speedup vs baseline: 1.0502x; 1.0502x over previous
import functools

import jax
import jax.numpy as jnp
import numpy as np
from jax import lax
from jax.experimental import pallas as pl
from jax.experimental.pallas import tpu as pltpu
from jax.experimental.pallas import tpu_sc as plsc

B, N = 4, 6
ogfH, ogfW = 256, 704
DS = 16
fH, fW = ogfH // DS, ogfW // DS
HW = fH * fW
D, C = 48, 64
NCAM = B * N
NX, NY = 200, 200
NVOX = NX * NY
VPAD = 40960
OFFX = np.float32(-50.0)
OFFY = np.float32(-50.0)
OFFZ = np.float32(-10.0)
DXX = np.float32(0.5)
DXY = np.float32(0.5)
DXZ = np.float32(20.0)
BXV = np.array([-49.75, -49.75, 0.0], dtype=np.float32)
DXV = np.array([0.5, 0.5, 20.0], dtype=np.float32)


def _create_frustum():
    ds = jnp.broadcast_to(jnp.linspace(2.0, 50.0, D, dtype=jnp.float32).reshape(D, 1, 1), (D, fH, fW))
    xs = jnp.broadcast_to(jnp.linspace(0.0, ogfW - 1.0, fW, dtype=jnp.float32).reshape(1, 1, fW), (D, fH, fW))
    ys = jnp.broadcast_to(jnp.linspace(0.0, ogfH - 1.0, fH, dtype=jnp.float32).reshape(1, fH, 1), (D, fH, fW))
    return jnp.stack([xs, ys, ds], axis=-1)


def _encode_body(img_ref, w_ref, b_ref, pool_ref, depth_ref, feat_ref):
    img = img_ref[0]
    a = jnp.sum(img.reshape(3, fH, DS, ogfW), axis=2)
    a2 = a.reshape(3 * fH, ogfW)
    xi44 = jnp.dot(a2, pool_ref[...], preferred_element_type=jnp.float32)
    xi3 = xi44.reshape(3, fH, fW)

    w = w_ref[...]
    bb = b_ref[...]
    out_d = (bb[0][:, None, None]
             + w[0][:, None, None] * xi3[0][None, :, :]
             + w[1][:, None, None] * xi3[1][None, :, :]
             + w[2][:, None, None] * xi3[2][None, :, :])
    logits = out_d[:D]
    m = jnp.max(logits, axis=0, keepdims=True)
    e = jnp.exp(logits - m)
    depth = e / jnp.sum(e, axis=0, keepdims=True)
    depth_ref[0] = depth

    out_c = (bb[0][None, None, :]
             + xi3[0][:, :, None] * w[0][None, None, :]
             + xi3[1][:, :, None] * w[1][None, None, :]
             + xi3[2][:, :, None] * w[2][None, None, :])
    feat_ref[0] = out_c[:, :, D:D + C]


def _encode(imgs, W_enc, b_enc, poolW):
    grid = (NCAM,)
    return pl.pallas_call(
        _encode_body,
        grid=grid,
        in_specs=[
            pl.BlockSpec((1, 3, ogfH, ogfW), lambda k: (k, 0, 0, 0)),
            pl.BlockSpec((3, D + C), lambda k: (0, 0)),
            pl.BlockSpec((1, D + C), lambda k: (0, 0)),
            pl.BlockSpec((ogfW, fW), lambda k: (0, 0)),
        ],
        out_specs=[
            pl.BlockSpec((1, D, fH, fW), lambda k: (k, 0, 0, 0)),
            pl.BlockSpec((1, fH, fW, C), lambda k: (k, 0, 0, 0)),
        ],
        out_shape=[
            jax.ShapeDtypeStruct((NCAM, D, fH, fW), jnp.float32),
            jax.ShapeDtypeStruct((NCAM, fH, fW, C), jnp.float32),
        ],
    )(imgs, W_enc, b_enc, poolW)


NSLAB = N * D
SPT = NSLAB // 16
RPT = VPAD // 16
FWORDS = HW * 16


def _bcast16(vec, l):
    dn = lax.GatherDimensionNumbers(offset_dims=(), collapsed_slice_dims=(0,),
                                    start_index_map=(0,))
    idxv = jnp.full((16,), l, jnp.int32)
    return lax.gather(vec, idxv[:, None], dn, (1,),
                      mode=lax.GatherScatterMode.PROMISE_IN_BOUNDS)


def _splat_body(depth_hbm, feat_hbm, idxp_hbm, zeros_hbm, out_hbm,
                featv, depv, ibuf, vbuf, zbuf, cbuf, tbuf, dsem, ssem, isem, wsem, acc):
    cid = lax.axis_index("c")
    tid = lax.axis_index("s")
    base = tid * RPT
    iota16 = lax.iota(jnp.int32, 16)
    pltpu.sync_copy(zeros_hbm, zbuf)
    pltpu.sync_copy(zeros_hbm.at[pl.ds(0, 64)], vbuf.at[pl.ds(704, 64)])

    def do_slab(j, b):
        s = b * NSLAB + tid * SPT + j
        h1 = pltpu.async_copy(depth_hbm.at[s], depv, dsem)
        h2 = pltpu.async_copy(idxp_hbm.at[s], ibuf, dsem)
        h1.wait()
        h2.wait()

        def grp(g, carry):
            dvec = depv[pl.ds(g * 16, 16)]
            for l in range(16):
                dj = _bcast16(dvec, l)
                r = g * 16 + l
                vbuf[r] = featv[pl.ds(r * 16, 16)] * dj
            return carry

        lax.fori_loop(0, HW // 16, grp, 0)
        hs = [pltpu.async_copy(vbuf.at[pl.ds(j6 * 128, 128)],
                               acc.at[ibuf.at[j6]], ssem, add=True)
              for j6 in range(6)]
        for hh in hs:
            hh.wait()
        return b

    def do_round(r, carry):
        b = cid * 2 + r // 4
        h = r % 4
        if True:
            z1 = pltpu.async_copy(zbuf, acc.at[pl.ds(base, 1280)], dsem)
            z2 = pltpu.async_copy(zbuf, acc.at[pl.ds(base + 1280, 1280)], dsem)
            z1.wait()
            z2.wait()
            plsc.subcore_barrier()
            s_lo = tid * SPT
            nA = s_lo // 48
            mA = jnp.minimum((nA + 1) * 48 - s_lo, SPT)
            pltpu.sync_copy(feat_hbm.at[h, b * N + nA], featv)
            lax.fori_loop(0, mA, do_slab, b)

            @pl.when(mA < SPT)
            def _():
                pltpu.sync_copy(feat_hbm.at[h, b * N + nA + 1], featv)

            lax.fori_loop(mA, SPT, do_slab, b)
            plsc.subcore_barrier()

            def xgath(c4, nrows):
                def xg(g, carry):
                    rowv = g * 16 + iota16
                    for cc in range(16):
                        vv = plsc.load_gather(cbuf.at[c4 % 2],
                                              [rowv, jnp.full((16,), cc, jnp.int32)])
                        tbuf[c4 % 2, cc, pl.ds(g * 16, 16)] = vv
                    return carry

                lax.fori_loop(0, nrows // 16, xg, 0)

            def in_dma(c4):
                return pltpu.make_async_copy(acc.at[pl.ds(base + c4 * 640, 640)],
                                             cbuf.at[c4 % 2], isem)

            def out_dma(c4):
                return pltpu.make_async_copy(
                    tbuf.at[c4 % 2],
                    out_hbm.at[b, pl.ds(h * 16, 16), pl.ds(base + c4 * 640, 640)],
                    wsem)

            def valid(c4):
                return base + (c4 + 1) * 640 <= NVOX

            for c4 in range(4):
                @pl.when(valid(c4))
                def _(c4=c4):
                    if c4 == 0:
                        in_dma(0).start()
                    in_dma(c4).wait()
                    if c4 < 3:
                        @pl.when(valid(c4 + 1))
                        def _():
                            in_dma(c4 + 1).start()
                    if c4 >= 2:
                        out_dma(c4 - 2).wait()
                    xgath(c4, 640)
                    out_dma(c4).start()

            for c4 in (2, 3):
                @pl.when(valid(c4))
                def _(c4=c4):
                    out_dma(c4).wait()

            @pl.when(tid == 15)
            def _():
                out_dma(0).wait()
                out_dma(1).wait()
                pltpu.sync_copy(acc.at[pl.ds(39680, 320)],
                                cbuf.at[0, pl.ds(0, 320)])

                def xg(g, carry):
                    rowv = g * 16 + iota16
                    for cc in range(16):
                        vv = plsc.load_gather(cbuf.at[0],
                                              [rowv, jnp.full((16,), cc, jnp.int32)])
                        tbuf[0, cc, pl.ds(g * 16, 16)] = vv
                    return carry

                lax.fori_loop(0, 20, xg, 0)
                pltpu.sync_copy(tbuf.at[0, :, pl.ds(0, 320)],
                                out_hbm.at[b, pl.ds(h * 16, 16), pl.ds(39680, 320)])

            plsc.subcore_barrier()
        return carry

    lax.fori_loop(0, 8, do_round, 0)


def _splat(depth_r, feat_r, idxp, zeros):
    mesh = plsc.VectorSubcoreMesh(core_axis_name="c", subcore_axis_name="s")
    f = pl.kernel(
        _splat_body,
        out_type=jax.ShapeDtypeStruct((B, C, NVOX), jnp.float32),
        mesh=mesh,
        scratch_types=[
            pltpu.VMEM((FWORDS,), jnp.float32),
            pltpu.VMEM((HW,), jnp.float32),
            pltpu.VMEM((6, 128), jnp.int32),
            pltpu.VMEM((768, 16), jnp.float32),
            pltpu.VMEM((1280, 16), jnp.float32),
            pltpu.VMEM((2, 640, 16), jnp.float32),
            pltpu.VMEM((2, 16, 640), jnp.float32),
            pltpu.SemaphoreType.DMA,
            pltpu.SemaphoreType.DMA,
            pltpu.SemaphoreType.DMA,
            pltpu.SemaphoreType.DMA,
            pltpu.VMEM_SHARED((VPAD, 16), jnp.float32),
        ],
        compiler_params=pltpu.CompilerParams(use_tc_tiling_on_sc=False, needs_layout_passes=False),
    )
    return f(depth_r, feat_r, idxp, zeros)


def kernel(imgs, rots, trans, intrins, post_rots, post_trans, W_enc, b_enc):
    frustum = _create_frustum()
    points = frustum[None, None] - post_trans.reshape(B, N, 1, 1, 1, 3)
    inv_post = jnp.linalg.inv(post_rots)
    points = jnp.einsum('bnij,bndhwj->bndhwi', inv_post, points)
    points = jnp.concatenate([points[..., :2] * points[..., 2:3], points[..., 2:3]], axis=-1)
    combine = jnp.einsum('bnij,bnjk->bnik', rots, jnp.linalg.inv(intrins))
    geom = jnp.einsum('bnij,bndhwj->bndhwi', combine, points) + trans.reshape(B, N, 1, 1, 1, 3)
    gf = ((geom - (jnp.asarray(BXV) - jnp.asarray(DXV) / 2.0)) / jnp.asarray(DXV)).astype(jnp.int32)
    ix, iy, iz = gf[..., 0], gf[..., 1], gf[..., 2]
    kept = (ix >= 0) & (ix < NX) & (iy >= 0) & (iy < NY) & (iz >= 0) & (iz < 1)
    lane = lax.broadcasted_iota(jnp.int32, (B, N, D, fH, fW), 4)
    idx = jnp.where(kept, iy * NX + ix, NVOX + (lane & 7))

    poolW = np.zeros((ogfW, fW), np.float32)
    for wq in range(fW):
        poolW[wq * DS:(wq + 1) * DS, wq] = 1.0 / (DS * DS)
    poolW = jnp.asarray(poolW)

    depth, feat = _encode(imgs.reshape(NCAM, 3, ogfH, ogfW), W_enc,
                          b_enc.reshape(1, D + C), poolW)

    depth_r = depth.reshape(NCAM * D, HW)
    feat_r = feat.reshape(NCAM, HW, 4, 16).transpose(2, 0, 1, 3).reshape(4, NCAM, FWORDS)
    idx_r = idx.reshape(NCAM * D, HW)
    tail = NVOX + (lax.broadcasted_iota(jnp.int32, (NCAM * D, 64), 1) & 7)
    idxp = jnp.concatenate([idx_r, tail], axis=1).reshape(NCAM * D, 6, 128)
    zeros = jnp.zeros((1280, 16), jnp.float32)
    out = _splat(depth_r, feat_r, idxp, zeros)
    return out.reshape(B, C, NY, NX)
